# baseline (speedup 1.0000x reference)
import jax
import jax.numpy as jnp
from jax import lax
from jax.experimental import pallas as pl
from jax.experimental.pallas import tpu as pltpu
from jax.experimental.pallas import tpu_sc as plsc

_D = 32
_B = 16384
_NC = 2
_NS = 16
_NW = _NC * _NS
_BPW = _B // _NW
_CHUNK = 128
_NCHUNK = _BPW // _CHUNK
_R = 100000
_VR = _R // 4
_NBLK = 782
_BLK_PER_W = 25


def _transpose_slab(blk_v, stg_v, width):
    row_lo = lax.iota(jnp.int32, 16)
    row_hi = row_lo + 16
    for q in range(width):
        col = jnp.full((16,), q, jnp.int32)
        e0 = plsc.load_gather(blk_v, [row_lo, col])
        e1 = plsc.load_gather(blk_v, [row_hi, col])
        stg_v[q // 4, pl.ds((q % 4) * 32, 16)] = e0
        stg_v[q // 4, pl.ds((q % 4) * 32 + 16, 16)] = e1


def _tbody(mu_t, lv_t, mu_tail, lv_tail, smu, slv, blk_v, stg_v):
    wid = lax.axis_index("s") * _NC + lax.axis_index("c")

    def blkloop(k, carry):
        c = k * _NW + wid

        @pl.when(c < _NBLK - 1)
        def _full():
            for src, dst in ((mu_t, smu), (lv_t, slv)):
                pltpu.sync_copy(src.at[:, pl.ds(c * 128, 128)], blk_v)
                _transpose_slab(blk_v, stg_v, 128)
                pltpu.sync_copy(stg_v, dst.at[pl.ds(c * 32, 32)])

        return carry

    lax.fori_loop(0, _BLK_PER_W, blkloop, 0)

    @pl.when(wid == 0)
    def _tail():
        for src, dst in ((mu_tail, smu), (lv_tail, slv)):
            pltpu.sync_copy(src, stg_v.at[pl.ds(0, 8)])
            pltpu.sync_copy(stg_v.at[pl.ds(0, 8)],
                            dst.at[pl.ds((_NBLK - 1) * 32, 8)])


def _gbody(idx_hbm, msk_hbm, smu, slv, muu_hbm, lvu_hbm,
           mu_out, lv_out,
           idx_v, vr_v, msk_v, muu_v, lvu_v, pad_v, st_mu, st_lv, sem):
    wid = lax.axis_index("s") * _NC + lax.axis_index("c")
    base = wid * _BPW

    pltpu.sync_copy(idx_hbm.at[pl.ds(base, _BPW)], idx_v)

    def mkvr(k, carry):
        vr_v[pl.ds(k * 16, 16)] = idx_v[pl.ds(k * 16, 16)] >> 2
        return carry
    lax.fori_loop(0, _BPW // 16, mkvr, 0)

    def gathers(tbl):
        return [
            pltpu.async_copy(tbl.at[vr_v.at[pl.ds(j * _CHUNK, _CHUNK)]],
                             pad_v.at[pl.ds(j * _CHUNK, _CHUNK)], sem)
            for j in range(_NCHUNK)
        ]

    mu_copies = gathers(smu)
    pltpu.sync_copy(msk_hbm.at[pl.ds(base, _BPW)], msk_v)
    pltpu.sync_copy(muu_hbm, muu_v)
    pltpu.sync_copy(lvu_hbm, lvu_v)
    mu_u = [muu_v[pl.ds(16 * t, 16)] for t in range(2)]
    lv_u = [lvu_v[pl.ds(16 * t, 16)] for t in range(2)]

    col_lo = lax.iota(jnp.int32, 16)
    col_hi = col_lo + 16

    def blend(st_v, u0, u1):
        def grp(g, carry):
            i16 = idx_v[pl.ds(g * 16, 16)]
            m16 = msk_v[pl.ds(g * 16, 16)]
            for r in range(16):
                su = (i16[r] & 3) * 32
                keep = m16[r] != 0
                i = g * 16 + r
                e0 = jnp.where(keep, pad_v[i, pl.ds(su, 16)], u0)
                e1 = jnp.where(keep, pad_v[i, pl.ds(su + 16, 16)], u1)
                item = jnp.full((16,), i, jnp.int32)
                plsc.store_scatter(st_v, [col_lo, item], e0)
                plsc.store_scatter(st_v, [col_hi, item], e1)
            return carry
        lax.fori_loop(0, _BPW // 16, grp, 0)

    for c in mu_copies:
        c.wait()
    blend(st_mu, mu_u[0], mu_u[1])
    lv_copies = gathers(slv)
    for c in lv_copies:
        c.wait()
    blend(st_lv, lv_u[0], lv_u[1])

    pltpu.sync_copy(st_mu, mu_out.at[:, pl.ds(base, _BPW)])
    pltpu.sync_copy(st_lv, lv_out.at[:, pl.ds(base, _BPW)])


def kernel(regime_id, regime_seen_mask, mu_embedding, logvar_embedding,
           mu_unknown, logvar_unknown):
    idx = regime_id.astype(jnp.int32)
    mesh = plsc.VectorSubcoreMesh(core_axis_name="c", subcore_axis_name="s")
    params = pltpu.CompilerParams(use_tc_tiling_on_sc=True,
                                  needs_layout_passes=False)

    tr = pl.kernel(
        _tbody,
        out_type=(jax.ShapeDtypeStruct((_VR, 128), jnp.float32),
                  jax.ShapeDtypeStruct((_VR, 128), jnp.float32)),
        mesh=mesh,
        compiler_params=params,
        scratch_types=[
            pltpu.VMEM((32, 128), jnp.float32),
            pltpu.VMEM((32, 128), jnp.float32),
        ],
    )
    mu_tail = mu_embedding[(_NBLK - 1) * 128:].reshape(8, 128)
    lv_tail = logvar_embedding[(_NBLK - 1) * 128:].reshape(8, 128)
    smu, slv = tr(mu_embedding.T, logvar_embedding.T, mu_tail, lv_tail)

    gt = pl.kernel(
        _gbody,
        out_type=(jax.ShapeDtypeStruct((_D, _B), jnp.float32),
                  jax.ShapeDtypeStruct((_D, _B), jnp.float32)),
        mesh=mesh,
        compiler_params=params,
        scratch_types=[
            pltpu.VMEM((_BPW,), jnp.int32),
            pltpu.VMEM((_BPW,), jnp.int32),
            pltpu.VMEM((_BPW,), jnp.int32),
            pltpu.VMEM((_D,), jnp.float32),
            pltpu.VMEM((_D,), jnp.float32),
            pltpu.VMEM((_BPW, 128), jnp.float32),
            pltpu.VMEM((_D, _BPW), jnp.float32),
            pltpu.VMEM((_D, _BPW), jnp.float32),
            pltpu.SemaphoreType.DMA,
        ],
    )
    o_mu_t, o_lv_t = gt(idx, regime_seen_mask, smu, slv,
                        mu_unknown, logvar_unknown)
    return (o_mu_t.T, o_lv_t.T)


# speedup vs baseline: 1.2994x; 1.2994x over previous
import jax
import jax.numpy as jnp
from jax import lax
from jax.experimental import pallas as pl
from jax.experimental.pallas import tpu as pltpu
from jax.experimental.pallas import tpu_sc as plsc

_D = 32
_B = 16384
_NC = 2
_NS = 16
_NW = _NC * _NS
_BPW = _B // _NW
_CHUNK = 128
_NCHUNK = _BPW // _CHUNK
_R = 100000
_VR = _R // 4
_NBLK = 782
_BLK_PER_W = 25


def _tbody(mu_t, lv_t, mu_tail, lv_tail, smu, slv, blk_v, stg_v,
           rsem, wsem):
    wid = lax.axis_index("s") * _NC + lax.axis_index("c")
    nblk = jnp.where(wid < 13, _BLK_PER_W, _BLK_PER_W - 1)

    def fire_reads(src, carry_unused):
        def rd(k, carry):
            c = k * _NW + wid
            cc = jnp.where(c < _NBLK - 1, c, wid)
            pltpu.async_copy(src.at[:, pl.ds(cc * 128, 128)],
                             blk_v.at[pl.ds(k * 32, 32)], rsem)
            return carry
        lax.fori_loop(0, _BLK_PER_W, rd, 0)

    def drain_one_read():
        pltpu.make_async_copy(mu_t.at[:, pl.ds(0, 128)],
                              blk_v.at[pl.ds(0, 32)], rsem).wait()

    def drain_one_store():
        pltpu.make_async_copy(stg_v.at[pl.ds(0, 32)],
                              smu.at[pl.ds(0, 32)], wsem).wait()

    row_lo = lax.iota(jnp.int32, 16)
    row_hi = row_lo + 16

    def process(dst, next_src, drain_from, carry_unused):
        def blk(k, carry):
            c = k * _NW + wid
            drain_one_read()

            @pl.when(k >= drain_from)
            def _():
                drain_one_store()

            p = (k & 3) * 32
            rb = k * 32

            def tq(q, carry2):
                col = jnp.full((16,), q, jnp.int32)
                e0 = plsc.load_gather(blk_v, [row_lo + rb, col])
                e1 = plsc.load_gather(blk_v, [row_hi + rb, col])
                stg_v[p + q // 4, pl.ds((q % 4) * 32, 16)] = e0
                stg_v[p + q // 4, pl.ds((q % 4) * 32 + 16, 16)] = e1
                return carry2

            lax.fori_loop(0, 128, tq, 0, unroll=8)

            @pl.when(c < _NBLK - 1)
            def _():
                pltpu.async_copy(stg_v.at[pl.ds(p, 32)],
                                 dst.at[pl.ds(c * 32, 32)], wsem)

            if next_src is not None:
                cc = jnp.where(c < _NBLK - 1, c, wid)
                pltpu.async_copy(next_src.at[:, pl.ds(cc * 128, 128)],
                                 blk_v.at[pl.ds(k * 32, 32)], rsem)
            return carry

        lax.fori_loop(0, _BLK_PER_W, blk, 0)
        def ep(k, carry):
            drain_one_store()
            return carry
        lax.fori_loop(0, nblk - 21, ep, 0)

    fire_reads(mu_t, None)
    process(smu, lv_t, 4, None)
    process(slv, None, 4, None)

    @pl.when(wid == 0)
    def _tail():
        for src, dst in ((mu_tail, smu), (lv_tail, slv)):
            pltpu.sync_copy(src, stg_v.at[pl.ds(0, 8)])
            pltpu.sync_copy(stg_v.at[pl.ds(0, 8)],
                            dst.at[pl.ds((_NBLK - 1) * 32, 8)])


def _gbody(idx_hbm, msk_hbm, smu, slv, muu_hbm, lvu_hbm,
           mu_out, lv_out,
           idx_v, vr_v, msk_v, muu_v, lvu_v, pad_v, st_mu, st_lv, sem):
    wid = lax.axis_index("s") * _NC + lax.axis_index("c")
    base = wid * _BPW

    pltpu.sync_copy(idx_hbm.at[pl.ds(base, _BPW)], idx_v)

    def mkvr(k, carry):
        vr_v[pl.ds(k * 16, 16)] = idx_v[pl.ds(k * 16, 16)] >> 2
        return carry
    lax.fori_loop(0, _BPW // 16, mkvr, 0)

    def gathers(tbl):
        return [
            pltpu.async_copy(tbl.at[vr_v.at[pl.ds(j * _CHUNK, _CHUNK)]],
                             pad_v.at[pl.ds(j * _CHUNK, _CHUNK)], sem)
            for j in range(_NCHUNK)
        ]

    mu_copies = gathers(smu)
    pltpu.sync_copy(msk_hbm.at[pl.ds(base, _BPW)], msk_v)
    pltpu.sync_copy(muu_hbm, muu_v)
    pltpu.sync_copy(lvu_hbm, lvu_v)
    mu_u = [muu_v[pl.ds(16 * t, 16)] for t in range(2)]
    lv_u = [lvu_v[pl.ds(16 * t, 16)] for t in range(2)]

    col_lo = lax.iota(jnp.int32, 16)
    col_hi = col_lo + 16

    def blend(st_v, u0, u1):
        def grp(g, carry):
            i16 = idx_v[pl.ds(g * 16, 16)]
            m16 = msk_v[pl.ds(g * 16, 16)]
            for r in range(16):
                su = (i16[r] & 3) * 32
                keep = m16[r] != 0
                i = g * 16 + r
                e0 = jnp.where(keep, pad_v[i, pl.ds(su, 16)], u0)
                e1 = jnp.where(keep, pad_v[i, pl.ds(su + 16, 16)], u1)
                item = jnp.full((16,), i, jnp.int32)
                plsc.store_scatter(st_v, [col_lo, item], e0)
                plsc.store_scatter(st_v, [col_hi, item], e1)
            return carry
        lax.fori_loop(0, _BPW // 16, grp, 0)

    for c in mu_copies:
        c.wait()
    blend(st_mu, mu_u[0], mu_u[1])
    lv_copies = gathers(slv)
    for c in lv_copies:
        c.wait()
    blend(st_lv, lv_u[0], lv_u[1])

    pltpu.sync_copy(st_mu, mu_out.at[:, pl.ds(base, _BPW)])
    pltpu.sync_copy(st_lv, lv_out.at[:, pl.ds(base, _BPW)])


def kernel(regime_id, regime_seen_mask, mu_embedding, logvar_embedding,
           mu_unknown, logvar_unknown):
    idx = regime_id.astype(jnp.int32)
    mesh = plsc.VectorSubcoreMesh(core_axis_name="c", subcore_axis_name="s")
    params = pltpu.CompilerParams(use_tc_tiling_on_sc=True,
                                  needs_layout_passes=False)

    tr = pl.kernel(
        _tbody,
        out_type=(jax.ShapeDtypeStruct((_VR, 128), jnp.float32),
                  jax.ShapeDtypeStruct((_VR, 128), jnp.float32)),
        mesh=mesh,
        compiler_params=params,
        scratch_types=[
            pltpu.VMEM((_BLK_PER_W * 32, 128), jnp.float32),
            pltpu.VMEM((4 * 32, 128), jnp.float32),
            pltpu.SemaphoreType.DMA,
            pltpu.SemaphoreType.DMA,
        ],
    )
    mu_tail = mu_embedding[(_NBLK - 1) * 128:].reshape(8, 128)
    lv_tail = logvar_embedding[(_NBLK - 1) * 128:].reshape(8, 128)
    smu, slv = tr(mu_embedding.T, logvar_embedding.T, mu_tail, lv_tail)

    gt = pl.kernel(
        _gbody,
        out_type=(jax.ShapeDtypeStruct((_D, _B), jnp.float32),
                  jax.ShapeDtypeStruct((_D, _B), jnp.float32)),
        mesh=mesh,
        compiler_params=params,
        scratch_types=[
            pltpu.VMEM((_BPW,), jnp.int32),
            pltpu.VMEM((_BPW,), jnp.int32),
            pltpu.VMEM((_BPW,), jnp.int32),
            pltpu.VMEM((_D,), jnp.float32),
            pltpu.VMEM((_D,), jnp.float32),
            pltpu.VMEM((_BPW, 128), jnp.float32),
            pltpu.VMEM((_D, _BPW), jnp.float32),
            pltpu.VMEM((_D, _BPW), jnp.float32),
            pltpu.SemaphoreType.DMA,
        ],
    )
    o_mu_t, o_lv_t = gt(idx, regime_seen_mask, smu, slv,
                        mu_unknown, logvar_unknown)
    return (o_mu_t.T, o_lv_t.T)


# speedup vs baseline: 2.0194x; 1.5540x over previous
import jax
import jax.numpy as jnp
from jax import lax
from jax.experimental import pallas as pl
from jax.experimental.pallas import tpu as pltpu
from jax.experimental.pallas import tpu_sc as plsc

_D = 32
_B = 16384
_NC = 2
_NS = 16
_NW = _NC * _NS
_BPW = _B // _NW
_CHUNK = 128
_NCHUNK = _BPW // _CHUNK
_STW = _BPW + 3


def _body(idx_hbm, msk_hbm, mu_hbm, lv_hbm, muu_hbm, lvu_hbm,
          mu_out, lv_out,
          idx_v, msk_v, muu_v, lvu_v, mu_rows, lv_rows, st_mu, st_lv, sem):
    wid = lax.axis_index("s") * _NC + lax.axis_index("c")
    base = wid * _BPW

    pltpu.sync_copy(idx_hbm.at[pl.ds(base, _BPW)], idx_v)
    copies = []
    for j in range(_NCHUNK):
        sl = pl.ds(j * _CHUNK, _CHUNK)
        copies.append(pltpu.async_copy(mu_hbm.at[idx_v.at[sl]],
                                       mu_rows.at[sl], sem))
        copies.append(pltpu.async_copy(lv_hbm.at[idx_v.at[sl]],
                                       lv_rows.at[sl], sem))
    pltpu.sync_copy(msk_hbm.at[pl.ds(base, _BPW)], msk_v)
    pltpu.sync_copy(muu_hbm, muu_v)
    pltpu.sync_copy(lvu_hbm, lvu_v)
    mu_u = [muu_v[pl.ds(16 * t, 16)] for t in range(2)]
    lv_u = [lvu_v[pl.ds(16 * t, 16)] for t in range(2)]
    for c in copies:
        c.wait()

    col_lo = lax.iota(jnp.int32, 16)
    col_hi = col_lo + 16

    def blend(g, carry):
        m16 = msk_v[pl.ds(g * 16, 16)]
        for r in range(16):
            i = g * 16 + r
            keep = m16[r] != 0
            item = jnp.full((16,), i, jnp.int32)
            e0 = jnp.where(keep, mu_rows[i, pl.ds(0, 16)], mu_u[0])
            e1 = jnp.where(keep, mu_rows[i, pl.ds(16, 16)], mu_u[1])
            plsc.store_scatter(st_mu, [col_lo, item], e0)
            plsc.store_scatter(st_mu, [col_hi, item], e1)
            f0 = jnp.where(keep, lv_rows[i, pl.ds(0, 16)], lv_u[0])
            f1 = jnp.where(keep, lv_rows[i, pl.ds(16, 16)], lv_u[1])
            plsc.store_scatter(st_lv, [col_lo, item], f0)
            plsc.store_scatter(st_lv, [col_hi, item], f1)
        return carry

    lax.fori_loop(0, _BPW // 16, blend, 0)

    pltpu.sync_copy(st_mu.at[:, pl.ds(0, _BPW)],
                    mu_out.at[:, pl.ds(base, _BPW)])
    pltpu.sync_copy(st_lv.at[:, pl.ds(0, _BPW)],
                    lv_out.at[:, pl.ds(base, _BPW)])


def kernel(regime_id, regime_seen_mask, mu_embedding, logvar_embedding,
           mu_unknown, logvar_unknown):
    idx = regime_id.astype(jnp.int32)
    mesh = plsc.VectorSubcoreMesh(core_axis_name="c", subcore_axis_name="s")
    f = pl.kernel(
        _body,
        out_type=(jax.ShapeDtypeStruct((_D, _B), jnp.float32),
                  jax.ShapeDtypeStruct((_D, _B), jnp.float32)),
        mesh=mesh,
        compiler_params=pltpu.CompilerParams(use_tc_tiling_on_sc=False,
                                             needs_layout_passes=False),
        scratch_types=[
            pltpu.VMEM((_BPW,), jnp.int32),
            pltpu.VMEM((_BPW,), jnp.int32),
            pltpu.VMEM((_D,), jnp.float32),
            pltpu.VMEM((_D,), jnp.float32),
            pltpu.VMEM((_BPW, _D), jnp.float32),
            pltpu.VMEM((_BPW, _D), jnp.float32),
            pltpu.VMEM((_D, _STW), jnp.float32),
            pltpu.VMEM((_D, _STW), jnp.float32),
            pltpu.SemaphoreType.DMA,
        ],
    )
    o_mu_t, o_lv_t = f(idx, regime_seen_mask, mu_embedding, logvar_embedding,
                       mu_unknown, logvar_unknown)
    return (o_mu_t.T, o_lv_t.T)


# speedup vs baseline: 2.1115x; 1.0456x over previous
import jax
import jax.numpy as jnp
from jax import lax
from jax.experimental import pallas as pl
from jax.experimental.pallas import tpu as pltpu
from jax.experimental.pallas import tpu_sc as plsc

_D = 32
_B = 16384
_NC = 2
_NS = 16
_NW = _NC * _NS
_BPW = _B // _NW
_CHUNK = 128
_NCHUNK = _BPW // _CHUNK
_STW = _BPW + 3


def _body(idx_hbm, msk_hbm, tbl_hbm, unk_hbm, out_hbm,
          idx_v, msk_v, unk_v, rows_v, st_v, sem):
    wid = lax.axis_index("s") * _NC + lax.axis_index("c")
    base = wid * _BPW

    pltpu.sync_copy(idx_hbm.at[pl.ds(base, _BPW)], idx_v)
    copies = [
        pltpu.async_copy(tbl_hbm.at[idx_v.at[pl.ds(j * _CHUNK, _CHUNK)]],
                         rows_v.at[pl.ds(j * _CHUNK, _CHUNK)], sem)
        for j in range(_NCHUNK)
    ]
    pltpu.sync_copy(msk_hbm.at[pl.ds(base, _BPW)], msk_v)
    pltpu.sync_copy(unk_hbm, unk_v)
    u0 = unk_v[pl.ds(0, 16)]
    u1 = unk_v[pl.ds(16, 16)]
    for c in copies:
        c.wait()

    col_lo = lax.iota(jnp.int32, 16)
    col_hi = col_lo + 16

    def blend(g, carry):
        m16 = msk_v[pl.ds(g * 16, 16)]
        for r in range(16):
            i = g * 16 + r
            keep = m16[r] != 0
            item = jnp.full((16,), i, jnp.int32)
            e0 = jnp.where(keep, rows_v[i, pl.ds(0, 16)], u0)
            e1 = jnp.where(keep, rows_v[i, pl.ds(16, 16)], u1)
            plsc.store_scatter(st_v, [col_lo, item], e0)
            plsc.store_scatter(st_v, [col_hi, item], e1)
        return carry

    lax.fori_loop(0, _BPW // 16, blend, 0)

    pltpu.sync_copy(st_v.at[:, pl.ds(0, _BPW)],
                    out_hbm.at[:, pl.ds(base, _BPW)])


def kernel(regime_id, regime_seen_mask, mu_embedding, logvar_embedding,
           mu_unknown, logvar_unknown):
    idx = regime_id.astype(jnp.int32)
    mesh = plsc.VectorSubcoreMesh(core_axis_name="c", subcore_axis_name="s")
    f = pl.kernel(
        _body,
        out_type=jax.ShapeDtypeStruct((_D, _B), jnp.float32),
        mesh=mesh,
        compiler_params=pltpu.CompilerParams(use_tc_tiling_on_sc=False,
                                             needs_layout_passes=False),
        scratch_types=[
            pltpu.VMEM((_BPW,), jnp.int32),
            pltpu.VMEM((_BPW,), jnp.int32),
            pltpu.VMEM((_D,), jnp.float32),
            pltpu.VMEM((_BPW, _D), jnp.float32),
            pltpu.VMEM((_D, _STW), jnp.float32),
            pltpu.SemaphoreType.DMA,
        ],
    )
    o_mu_t = f(idx, regime_seen_mask, mu_embedding, mu_unknown)
    o_lv_t = f(idx, regime_seen_mask, logvar_embedding, logvar_unknown)
    return (o_mu_t.T, o_lv_t.T)
